# baseline (speedup 1.0000x reference)
import functools
import jax
import jax.numpy as jnp
from jax import lax
from jax.experimental import pallas as pl
from jax.experimental.pallas import tpu as pltpu
from jax.experimental.pallas import tpu_sc as plsc

_B = 1024
_V = 200
_U1 = 101
_LANES = 16
_NCH = 7
_UP = _NCH * _LANES
_ROW = _V * _U1
_GROW = _V * _UP
_NW = 32
_BPW = _B // _NW
_C6 = 85


def _tec_kernel(w_hbm, g_hbm, t_hbm, ltab_hbm, sizes_hbm, seq_hbm,
                w_a, g_a, w_b, g_b, t_v, ltab_v, sizes_v, seq_v,
                sem_wa, sem_ga, sem_wb, sem_gb):
    cid = lax.axis_index("c")
    sid = lax.axis_index("s")
    wid = sid * 2 + cid
    base = wid * _BPW

    pltpu.sync_copy(t_hbm.at[pl.ds(base, _BPW)], t_v)
    pltpu.sync_copy(ltab_hbm, ltab_v)

    iota = lax.iota(jnp.int32, _LANES)
    lane0 = iota == 0
    mask6 = iota >= 11
    perms = [(iota ^ s)[:, None] for s in (8, 4, 2, 1)]
    dnums = lax.GatherDimensionNumbers(
        offset_dims=(), collapsed_slice_dims=(0,), start_index_map=(0,))

    def shuf(x, p):
        return lax.gather(x, p, dnums, slice_sizes=(1,),
                          mode=lax.GatherScatterMode.PROMISE_IN_BOUNDS)

    zero_i = jnp.zeros((_LANES,), jnp.int32)
    zero_f = jnp.zeros((_LANES,), jnp.float32)
    inf_f = jnp.full((_LANES,), 3.0e38, jnp.float32)
    neg_big = jnp.full((_LANES,), -1e30, jnp.float32)

    def start(b, w_buf, g_buf, sem_w, sem_g):
        pltpu.make_async_copy(
            w_hbm.at[pl.ds(b * _ROW, _ROW)], w_buf, sem_w).start()
        pltpu.make_async_copy(
            g_hbm.at[pl.ds(b * _GROW, _GROW)], g_buf, sem_g).start()

    def wait(b, w_buf, g_buf, sem_w, sem_g):
        pltpu.make_async_copy(
            w_hbm.at[pl.ds(b * _ROW, _ROW)], w_buf, sem_w).wait()
        pltpu.make_async_copy(
            g_hbm.at[pl.ds(b * _GROW, _GROW)], g_buf, sem_g).wait()

    def process(bl, w_v, g_v):
        t_b = plsc.load_gather(t_v, [jnp.full((_LANES,), bl, jnp.int32)])

        def v_body(v, carry):
            size_vec = carry[0]
            talive = carry[1:]
            woff = v * _U1
            goff = v * _UP
            cand = []
            gc = []
            for i in range(_NCH):
                ws = woff + (i * _LANES if i < 6 else _C6)
                w_i = w_v[pl.ds(ws, _LANES)]
                g_i = g_v[pl.ds(goff + i * _LANES, _LANES)]
                c_i = w_i >= talive[i]
                if i == 6:
                    c_i = c_i & mask6
                cand.append(c_i)
                gc.append(jnp.where(c_i, g_i, neg_big))

            kv = plsc.all_reduce_population_count(cand[0])
            for i in range(1, _NCH):
                kv = kv + plsc.all_reduce_population_count(cand[i])
            l_k = plsc.load_gather(ltab_v, [kv])

            m01 = jnp.maximum(gc[0], gc[1])
            m23 = jnp.maximum(gc[2], gc[3])
            m45 = jnp.maximum(gc[4], gc[5])
            m = jnp.maximum(jnp.maximum(m01, m23), jnp.maximum(m45, gc[6]))
            for p in perms:
                m = jnp.maximum(m, shuf(m, p))
            big_m = m + l_k

            idxs = [jnp.where(gc[i] + l_k == big_m,
                              iota + (i * _LANES if i < 6 else _C6),
                              jnp.full((_LANES,), 10000, jnp.int32))
                    for i in range(_NCH)]
            i01 = jnp.minimum(idxs[0], idxs[1])
            i23 = jnp.minimum(idxs[2], idxs[3])
            i45 = jnp.minimum(idxs[4], idxs[5])
            sel = jnp.minimum(jnp.minimum(i01, i23),
                              jnp.minimum(i45, idxs[6]))
            for p in perms:
                sel = jnp.minimum(sel, shuf(sel, p))

            w_sel = plsc.load_gather(w_v, [woff + sel])
            size_vec = size_vec + w_sel

            hit = sel > 0
            new_talive = tuple(
                jnp.where((iota + (i * _LANES if i < 6 else _C6) == sel)
                          & hit, inf_f, talive[i])
                for i in range(_NCH))
            plsc.store_scatter(seq_v, [jnp.full((_LANES,), v, jnp.int32)],
                               sel, mask=lane0)
            return (size_vec,) + new_talive

        def v_body2(h, carry):
            carry = v_body(2 * h, carry)
            return v_body(2 * h + 1, carry)

        init = (zero_f,) + tuple(t_b for _ in range(_NCH))
        final = lax.fori_loop(0, _V // 2, v_body2, init)
        plsc.store_scatter(sizes_v, [jnp.full((_LANES,), bl, jnp.int32)],
                           final[0], mask=lane0)

    start(base, w_a, g_a, sem_wa, sem_ga)

    def b_body(j, _):
        b0 = base + 2 * j
        b1 = b0 + 1
        b2 = jnp.minimum(b0 + 2, _B - 1)
        wait(b0, w_a, g_a, sem_wa, sem_ga)
        start(b1, w_b, g_b, sem_wb, sem_gb)
        process(2 * j, w_a, g_a)
        pltpu.sync_copy(seq_v, seq_hbm.at[b0])
        wait(b1, w_b, g_b, sem_wb, sem_gb)
        start(b2, w_a, g_a, sem_wa, sem_ga)
        process(2 * j + 1, w_b, g_b)
        pltpu.sync_copy(seq_v, seq_hbm.at[b1])
        return 0

    lax.fori_loop(0, _BPW // 2, b_body, 0)
    wait(_B - 1, w_a, g_a, sem_wa, sem_ga)
    pltpu.sync_copy(sizes_v, sizes_hbm.at[pl.ds(base, _BPW)])


@jax.jit
def _run(w1, g1, t1, ltab):
    mesh = plsc.VectorSubcoreMesh(core_axis_name="c", subcore_axis_name="s",
                                  num_cores=2, num_subcores=16)
    f = pl.kernel(
        _tec_kernel,
        out_type=[
            jax.ShapeDtypeStruct((_B,), jnp.float32),
            jax.ShapeDtypeStruct((_B, _V), jnp.int32),
        ],
        mesh=mesh,
        compiler_params=pltpu.CompilerParams(needs_layout_passes=False),
        scratch_types=[
            pltpu.VMEM((_ROW,), jnp.float32),
            pltpu.VMEM((_GROW,), jnp.float32),
            pltpu.VMEM((_ROW,), jnp.float32),
            pltpu.VMEM((_GROW,), jnp.float32),
            pltpu.VMEM((_BPW,), jnp.float32),
            pltpu.VMEM((104,), jnp.float32),
            pltpu.VMEM((_BPW,), jnp.float32),
            pltpu.VMEM((_V,), jnp.int32),
            pltpu.SemaphoreType.DMA,
            pltpu.SemaphoreType.DMA,
            pltpu.SemaphoreType.DMA,
            pltpu.SemaphoreType.DMA,
        ],
    )
    return f(w1, g1, t1, ltab)


def kernel(weights, t):
    B, V, U1 = weights.shape
    with jax.ensure_compile_time_eval():
        sample_key = jax.random.key(42)
        keys = jax.vmap(
            lambda v: jax.random.fold_in(sample_key, v))(jnp.arange(V))
        gum = jax.vmap(
            lambda k: jax.random.gumbel(k, (B, U1), jnp.float32))(keys)
        gum = jnp.transpose(gum, (1, 0, 2))
        g1 = jnp.concatenate(
            [gum[:, :, :96], gum[:, :, _C6:]], axis=2).reshape(-1)
        counts = jnp.arange(104, dtype=jnp.float32).at[0].set(1.0)
        ltab = jnp.log(1.0 / counts)
    sizes, seq = _run(weights.reshape(-1), g1, t.reshape(B), ltab)
    return (-sizes / V, seq.reshape(B, V, 1))


# speedup vs baseline: 1.2575x; 1.2575x over previous
import functools
import jax
import jax.numpy as jnp
from jax import lax
from jax.experimental import pallas as pl
from jax.experimental.pallas import tpu as pltpu
from jax.experimental.pallas import tpu_sc as plsc

_B = 1024
_V = 200
_U1 = 101
_LANES = 16
_NCH = 7
_UP = _NCH * _LANES
_ROW = _V * _U1
_GROW = _V * _UP
_NW = 32
_BPW = _B // _NW
_C6 = 85


def _tec_kernel(w_hbm, g_hbm, t_hbm, ltab_hbm, sizes_hbm, seq_hbm,
                w_a, g_a, w_b, g_b, t_v, ltab_v, sizes_v, seq_v,
                sem_wa, sem_ga, sem_wb, sem_gb):
    cid = lax.axis_index("c")
    sid = lax.axis_index("s")
    wid = sid * 2 + cid
    base = wid * _BPW

    pltpu.sync_copy(t_hbm.at[pl.ds(base, _BPW)], t_v)
    pltpu.sync_copy(ltab_hbm, ltab_v)

    iota = lax.iota(jnp.int32, _LANES)
    lane0 = iota == 0
    mask6 = iota >= 11
    perms = [(iota ^ s)[:, None] for s in (8, 4, 2, 1)]
    dnums = lax.GatherDimensionNumbers(
        offset_dims=(), collapsed_slice_dims=(0,), start_index_map=(0,))

    def shuf(x, p):
        return lax.gather(x, p, dnums, slice_sizes=(1,),
                          mode=lax.GatherScatterMode.PROMISE_IN_BOUNDS)

    zero_i = jnp.zeros((_LANES,), jnp.int32)
    zero_f = jnp.zeros((_LANES,), jnp.float32)
    inf_f = jnp.full((_LANES,), 3.0e38, jnp.float32)
    neg_big = jnp.full((_LANES,), -1e30, jnp.float32)

    def start(b, w_buf, g_buf, sem_w, sem_g):
        pltpu.make_async_copy(w_hbm.at[b], w_buf, sem_w).start()
        pltpu.make_async_copy(
            g_hbm.at[pl.ds(b * _GROW, _GROW)], g_buf, sem_g).start()

    def wait(b, w_buf, g_buf, sem_w, sem_g):
        pltpu.make_async_copy(w_hbm.at[b], w_buf, sem_w).wait()
        pltpu.make_async_copy(
            g_hbm.at[pl.ds(b * _GROW, _GROW)], g_buf, sem_g).wait()

    def process(bl, w_v, g_v):
        t_b = plsc.load_gather(t_v, [jnp.full((_LANES,), bl, jnp.int32)])

        def v_body(v, carry):
            size_vec = carry[0]
            talive = carry[1:]
            goff = v * _UP
            cand = []
            gc = []
            for i in range(_NCH):
                ws = i * _LANES if i < 6 else _C6
                w_i = w_v[v, pl.ds(ws, _LANES)]
                g_i = g_v[pl.ds(goff + i * _LANES, _LANES)]
                c_i = w_i >= talive[i]
                if i == 6:
                    c_i = c_i & mask6
                cand.append(c_i)
                gc.append(jnp.where(c_i, g_i, neg_big))

            kv = plsc.all_reduce_population_count(cand[0])
            for i in range(1, _NCH):
                kv = kv + plsc.all_reduce_population_count(cand[i])
            l_k = plsc.load_gather(ltab_v, [kv])

            m01 = jnp.maximum(gc[0], gc[1])
            m23 = jnp.maximum(gc[2], gc[3])
            m45 = jnp.maximum(gc[4], gc[5])
            m = jnp.maximum(jnp.maximum(m01, m23), jnp.maximum(m45, gc[6]))
            for p in perms:
                m = jnp.maximum(m, shuf(m, p))
            big_m = m + l_k

            idxs = [jnp.where(gc[i] + l_k == big_m,
                              iota + (i * _LANES if i < 6 else _C6),
                              jnp.full((_LANES,), 10000, jnp.int32))
                    for i in range(_NCH)]
            i01 = jnp.minimum(idxs[0], idxs[1])
            i23 = jnp.minimum(idxs[2], idxs[3])
            i45 = jnp.minimum(idxs[4], idxs[5])
            sel = jnp.minimum(jnp.minimum(i01, i23),
                              jnp.minimum(i45, idxs[6]))
            for p in perms:
                sel = jnp.minimum(sel, shuf(sel, p))

            w_sel = plsc.load_gather(
                w_v, [jnp.full((_LANES,), v, jnp.int32), sel])
            size_vec = size_vec + w_sel

            hit = sel > 0
            new_talive = tuple(
                jnp.where((iota + (i * _LANES if i < 6 else _C6) == sel)
                          & hit, inf_f, talive[i])
                for i in range(_NCH))
            plsc.store_scatter(seq_v, [jnp.full((_LANES,), v, jnp.int32)],
                               sel, mask=lane0)
            return (size_vec,) + new_talive

        init = (zero_f,) + tuple(t_b for _ in range(_NCH))
        final = lax.fori_loop(0, _V, v_body, init)
        plsc.store_scatter(sizes_v, [jnp.full((_LANES,), bl, jnp.int32)],
                           final[0], mask=lane0)

    start(base, w_a, g_a, sem_wa, sem_ga)

    def b_body(j, _):
        b0 = base + 2 * j
        b1 = b0 + 1
        b2 = jnp.minimum(b0 + 2, _B - 1)
        wait(b0, w_a, g_a, sem_wa, sem_ga)
        start(b1, w_b, g_b, sem_wb, sem_gb)
        process(2 * j, w_a, g_a)
        pltpu.sync_copy(seq_v, seq_hbm.at[pl.ds(b0 * _V, _V)])
        wait(b1, w_b, g_b, sem_wb, sem_gb)
        start(b2, w_a, g_a, sem_wa, sem_ga)
        process(2 * j + 1, w_b, g_b)
        pltpu.sync_copy(seq_v, seq_hbm.at[pl.ds(b1 * _V, _V)])
        return 0

    lax.fori_loop(0, _BPW // 2, b_body, 0)
    wait(_B - 1, w_a, g_a, sem_wa, sem_ga)
    pltpu.sync_copy(sizes_v, sizes_hbm.at[pl.ds(base, _BPW)])


@jax.jit
def _run(w1, g1, t1, ltab):
    mesh = plsc.VectorSubcoreMesh(core_axis_name="c", subcore_axis_name="s",
                                  num_cores=2, num_subcores=16)
    f = pl.kernel(
        _tec_kernel,
        out_type=[
            jax.ShapeDtypeStruct((_B,), jnp.float32),
            jax.ShapeDtypeStruct((_B * _V,), jnp.int32),
        ],
        mesh=mesh,
        compiler_params=pltpu.CompilerParams(needs_layout_passes=False,
                                             use_tc_tiling_on_sc=True),
        scratch_types=[
            pltpu.VMEM((_V, _U1), jnp.float32),
            pltpu.VMEM((_GROW,), jnp.float32),
            pltpu.VMEM((_V, _U1), jnp.float32),
            pltpu.VMEM((_GROW,), jnp.float32),
            pltpu.VMEM((_BPW,), jnp.float32),
            pltpu.VMEM((104,), jnp.float32),
            pltpu.VMEM((_BPW,), jnp.float32),
            pltpu.VMEM((_V,), jnp.int32),
            pltpu.SemaphoreType.DMA,
            pltpu.SemaphoreType.DMA,
            pltpu.SemaphoreType.DMA,
            pltpu.SemaphoreType.DMA,
        ],
    )
    return f(w1, g1, t1, ltab)


def kernel(weights, t):
    B, V, U1 = weights.shape
    with jax.ensure_compile_time_eval():
        sample_key = jax.random.key(42)
        keys = jax.vmap(
            lambda v: jax.random.fold_in(sample_key, v))(jnp.arange(V))
        gum = jax.vmap(
            lambda k: jax.random.gumbel(k, (B, U1), jnp.float32))(keys)
        gum = jnp.transpose(gum, (1, 0, 2))
        g1 = jnp.concatenate(
            [gum[:, :, :96], gum[:, :, _C6:]], axis=2).reshape(-1)
        counts = jnp.arange(104, dtype=jnp.float32).at[0].set(1.0)
        ltab = jnp.log(1.0 / counts)
    sizes, seq = _run(weights, g1, t.reshape(B), ltab)
    return (-sizes / V, seq.reshape(B, V, 1))
